# baseline (speedup 1.0000x reference)
import jax
import jax.numpy as jnp
from jax import lax
from jax.experimental import pallas as pl
from jax.experimental.pallas import tpu as pltpu
from jax.experimental.pallas import tpu_sc as plsc

N, E, C, D = 59, 944, 3, 9
NPAD = 64
EPAD = 1024
NSUB = 16
GPS = EPAD // 16 // NSUB
EPS = EPAD // NSUB
ZD = 2 * C + D
FLAT = C * NPAD
RW = FLAT // 16
FLATP = 256
EB = (2 + D) * EPS

PX, PWF, PWS, PBF, PBS, PWL, PBL = 0, 192, 912, 1632, 1680, 1728, 1920
PTOT = 1936

f32 = jnp.float32


def _exp(v):
    v = jnp.minimum(jnp.maximum(v, -87.0), 87.0)
    t = v * 1.4426950408889634
    n_i = (t + 0.5 * jnp.sign(t)).astype(jnp.int32)
    n_f = n_i.astype(f32)
    r = v - n_f * 0.6931471824645996 - n_f * (-1.904654323148236e-09)
    p = 1.0 + r * (1.0 + r * (0.5 + r * (1.0 / 6.0 + r * (1.0 / 24.0 + r * (1.0 / 120.0 + r * (1.0 / 720.0))))))
    scale = plsc.bitcast(jnp.left_shift(n_i + 127, 23), f32)
    return p * scale


def _bf16r(v):
    u = plsc.bitcast(v, jnp.int32)
    lsb = jnp.bitwise_and(jnp.right_shift(u, 16), 1)
    u = u + 0x7FFF + lsb
    u = jnp.bitwise_and(u, jnp.int32(-65536))
    return plsc.bitcast(u, f32)


def _div(a, b):
    r = 1.0 / b
    r = r * (2.0 - b * r)
    return a * r


def _sigmoid(v):
    return _div(1.0, 1.0 + _exp(-v))


def _softplus(v):
    a = _exp(-jnp.abs(v))
    z = _div(a, a + 2.0)
    z2 = z * z
    p = z * (1.0 + z2 * (1.0 / 3.0 + z2 * (0.2 + z2 * (1.0 / 7.0 + z2 * (1.0 / 9.0)))))
    return jnp.maximum(v, 0.0) + 2.0 * p


def _body(par_hbm, edg_hbm, zidx_hbm, out_hbm,
          par_v, edg_v, agg_v, fin_v, out_v, idx1, shared, sem):
    sid = lax.axis_index("s")

    cp_par = pltpu.async_copy(par_hbm, par_v, sem)
    cp_edg = pltpu.async_copy(edg_hbm.at[pl.ds(sid * EB, EB)], edg_v, sem)
    cp_idx = pltpu.async_copy(zidx_hbm.at[pl.ds(0, 1)], idx1, sem)
    cp_par.wait()
    cp_edg.wait()
    cp_idx.wait()

    zero = jnp.zeros((16,), f32)
    zrow = jnp.zeros((16,), jnp.int32)
    for r in range(FLATP // 16):
        agg_v[0, pl.ds(r * 16, 16)] = zero

    @pl.when(sid == 0)
    def _():
        pltpu.sync_copy(agg_v, shared)
    plsc.subcore_barrier()

    for j in range(GPS):
        off = j * 16
        src16 = edg_v[pl.ds(off, 16)]
        dst16 = edg_v[pl.ds(EPS + off, 16)]
        feats = []
        for c in range(C):
            feats.append(plsc.load_gather(par_v, [dst16 + (PX + c * NPAD)]))
        for c in range(C):
            feats.append(plsc.load_gather(par_v, [src16 + (PX + c * NPAD)]))
        for k in range(D):
            feats.append(plsc.bitcast(edg_v[pl.ds((2 + k) * EPS + off, 16)], f32))
        for c in range(C):
            f = par_v[pl.ds(PBF + c * 16, 16)]
            s = par_v[pl.ds(PBS + c * 16, 16)]
            for k in range(ZD):
                f = f + feats[k] * par_v[pl.ds(PWF + (c * ZD + k) * 16, 16)]
                s = s + feats[k] * par_v[pl.ds(PWS + (c * ZD + k) * 16, 16)]
            m = _sigmoid(f) * _softplus(s)
            plsc.addupdate_scatter(agg_v, [zrow, dst16 + (c * NPAD)], m)

    pltpu.sync_copy(agg_v, shared.at[idx1], add=True)

    plsc.subcore_barrier()

    @pl.when(sid == 0)
    def _():
        pltpu.sync_copy(shared.at[0], fin_v)
        acc = par_v[pl.ds(PBL, 16)]
        for r in range(RW):
            sl = pl.ds(r * 16, 16)
            aggr = fin_v[sl]
            xr = par_v[pl.ds(PX + r * 16, 16)]
            wr = par_v[pl.ds(PWL + r * 16, 16)]
            acc = acc + jnp.maximum(xr + aggr, 0.0) * wr
        out_v[...] = jnp.cumsum(acc)
        pltpu.sync_copy(out_v, out_hbm)


@jax.jit
def _run(par, edg):
    mesh = plsc.VectorSubcoreMesh(core_axis_name="c", subcore_axis_name="s",
                                  num_cores=1)
    kfn = pl.kernel(
        _body,
        out_type=jax.ShapeDtypeStruct((16,), f32),
        mesh=mesh,
        scratch_types=[
            pltpu.VMEM((PTOT,), f32),
            pltpu.VMEM((EB,), jnp.int32),
            pltpu.VMEM((1, FLATP), f32),
            pltpu.VMEM((FLATP,), f32),
            pltpu.VMEM((16,), f32),
            pltpu.VMEM((1,), jnp.int32),
            pltpu.VMEM_SHARED((1, FLATP), f32),
            pltpu.SemaphoreType.DMA,
        ],
        compiler_params=pltpu.CompilerParams(needs_layout_passes=False),
    )
    return kfn(par, edg, jnp.zeros((8,), jnp.int32))


def kernel(x, edge_index, edge_attr, batch, Wf, bf, Ws, bs, Wl, bl):
    xt = jnp.zeros((C, NPAD), f32).at[:, :N].set(x.T).reshape(FLAT)
    def bf16r_host(v):
        u = lax.bitcast_convert_type(v, jnp.int32)
        u = u + 0x7FFF + jnp.bitwise_and(jnp.right_shift(u, 16), 1)
        u = jnp.bitwise_and(u, jnp.int32(-65536))
        return lax.bitcast_convert_type(u, f32)

    wfr = bf16r_host(Wf)
    wsr = bf16r_host(Ws)
    wfb = jnp.broadcast_to(wfr[:, :, None], (C, ZD, 16)).reshape(-1)
    wsb = jnp.broadcast_to(wsr[:, :, None], (C, ZD, 16)).reshape(-1)
    bfb = jnp.broadcast_to(bf.reshape(C, 1), (C, 16)).reshape(-1)
    bsb = jnp.broadcast_to(bs.reshape(C, 1), (C, 16)).reshape(-1)
    wl2 = jnp.zeros((C, NPAD), f32).at[:, :N].set(Wl[0].reshape(N, C).T).reshape(FLAT)
    blb = jnp.zeros((16,), f32).at[0].set(bl[0])
    par = jnp.concatenate([xt, wfb, wsb, bfb, bsb, wl2, blb])

    srcp = jnp.zeros((EPAD,), jnp.int32).at[:E].set(edge_index[0]).reshape(NSUB, 1, EPS)
    dstp = jnp.full((EPAD,), N, jnp.int32).at[:E].set(edge_index[1]).reshape(NSUB, 1, EPS)
    attr_t = jnp.zeros((D, EPAD), f32).at[:, :E].set(edge_attr.T)
    attr_i = lax.bitcast_convert_type(attr_t, jnp.int32)
    attr_b = attr_i.reshape(D, NSUB, EPS).transpose(1, 0, 2)
    edg = jnp.concatenate([srcp, dstp, attr_b], axis=1).reshape(-1)

    out16 = _run(par, edg)
    return out16[15:16]


# speedup vs baseline: 1.0122x; 1.0122x over previous
import jax
import jax.numpy as jnp
from jax import lax
from jax.experimental import pallas as pl
from jax.experimental.pallas import tpu as pltpu
from jax.experimental.pallas import tpu_sc as plsc

N, E, C, D = 59, 944, 3, 9
NPAD = 64
EPAD = 1024
NSUB = 16
GPS = EPAD // 16 // NSUB
EPS = EPAD // NSUB
ZD = 2 * C + D
FLAT = C * NPAD
RW = FLAT // 16
FLATP = 256
EB = (2 + D) * EPS

PX, PWF, PWS, PBF, PBS, PWL, PBL = 0, 192, 912, 1632, 1680, 1728, 1920
PTOT = 1936

f32 = jnp.float32


def _exp(v):
    v = jnp.minimum(jnp.maximum(v, -87.0), 87.0)
    t = v * 1.4426950408889634
    n_i = (t + 0.5 * jnp.sign(t)).astype(jnp.int32)
    n_f = n_i.astype(f32)
    r = v - n_f * 0.6931471824645996 - n_f * (-1.904654323148236e-09)
    p = 1.0 + r * (1.0 + r * (0.5 + r * (1.0 / 6.0 + r * (1.0 / 24.0 + r * (1.0 / 120.0 + r * (1.0 / 720.0))))))
    scale = plsc.bitcast(jnp.left_shift(n_i + 127, 23), f32)
    return p * scale


def _bf16r(v):
    u = plsc.bitcast(v, jnp.int32)
    lsb = jnp.bitwise_and(jnp.right_shift(u, 16), 1)
    u = u + 0x7FFF + lsb
    u = jnp.bitwise_and(u, jnp.int32(-65536))
    return plsc.bitcast(u, f32)


def _div(a, b):
    r = 1.0 / b
    r = r * (2.0 - b * r)
    return a * r


def _sigmoid(v):
    return _div(1.0, 1.0 + _exp(-v))


def _softplus(v):
    a = _exp(-jnp.abs(v))
    z = _div(a, a + 2.0)
    z2 = z * z
    p = z * (1.0 + z2 * (1.0 / 3.0 + z2 * (0.2 + z2 * (1.0 / 7.0 + z2 * (1.0 / 9.0)))))
    return jnp.maximum(v, 0.0) + 2.0 * p


def _body(par_hbm, edg_hbm, out_hbm,
          par_v, edg_v, agg_v, part_v, out_v, shared, sem):
    sid = lax.axis_index("s")

    cp_par = pltpu.async_copy(par_hbm, par_v, sem)
    cp_edg = pltpu.async_copy(edg_hbm.at[pl.ds(sid * EB, EB)], edg_v, sem)
    cp_par.wait()
    cp_edg.wait()

    zero = jnp.zeros((16,), f32)
    for r in range(FLATP // 16):
        agg_v[pl.ds(r * 16, 16)] = zero

    for j in range(GPS):
        off = j * 16
        src16 = edg_v[pl.ds(off, 16)]
        dst16 = edg_v[pl.ds(EPS + off, 16)]
        feats = []
        for c in range(C):
            feats.append(plsc.load_gather(par_v, [dst16 + (PX + c * NPAD)]))
        for c in range(C):
            feats.append(plsc.load_gather(par_v, [src16 + (PX + c * NPAD)]))
        for k in range(D):
            feats.append(plsc.bitcast(edg_v[pl.ds((2 + k) * EPS + off, 16)], f32))
        for c in range(C):
            f = par_v[pl.ds(PBF + c * 16, 16)]
            s = par_v[pl.ds(PBS + c * 16, 16)]
            for k in range(ZD):
                f = f + feats[k] * par_v[pl.ds(PWF + (c * ZD + k) * 16, 16)]
                s = s + feats[k] * par_v[pl.ds(PWS + (c * ZD + k) * 16, 16)]
            m = _sigmoid(f) * _softplus(s)
            plsc.addupdate_scatter(agg_v, [dst16 + (c * NPAD)], m)

    pltpu.sync_copy(agg_v, shared.at[sid])

    plsc.subcore_barrier()

    @pl.when(sid == 0)
    def _():
        rcps = [
            pltpu.async_copy(shared.at[p], part_v.at[pl.ds(p * FLATP, FLATP)], sem)
            for p in range(NSUB)
        ]
        for cp in rcps:
            cp.wait()
        acc = par_v[pl.ds(PBL, 16)]
        for r in range(RW):
            sl = pl.ds(r * 16, 16)
            aggr = part_v[sl]
            for p in range(1, NSUB):
                aggr = aggr + part_v[pl.ds(p * FLATP + r * 16, 16)]
            xr = par_v[pl.ds(PX + r * 16, 16)]
            wr = par_v[pl.ds(PWL + r * 16, 16)]
            acc = acc + jnp.maximum(xr + aggr, 0.0) * wr
        out_v[...] = jnp.cumsum(acc)
        pltpu.sync_copy(out_v, out_hbm)


@jax.jit
def _run(par, edg):
    mesh = plsc.VectorSubcoreMesh(core_axis_name="c", subcore_axis_name="s",
                                  num_cores=1)
    kfn = pl.kernel(
        _body,
        out_type=jax.ShapeDtypeStruct((16,), f32),
        mesh=mesh,
        scratch_types=[
            pltpu.VMEM((PTOT,), f32),
            pltpu.VMEM((EB,), jnp.int32),
            pltpu.VMEM((FLATP,), f32),
            pltpu.VMEM((NSUB * FLATP,), f32),
            pltpu.VMEM((16,), f32),
            pltpu.VMEM_SHARED((NSUB, FLATP), f32),
            pltpu.SemaphoreType.DMA,
        ],
        compiler_params=pltpu.CompilerParams(needs_layout_passes=False),
    )
    return kfn(par, edg)


def kernel(x, edge_index, edge_attr, batch, Wf, bf, Ws, bs, Wl, bl):
    xt = jnp.zeros((C, NPAD), f32).at[:, :N].set(x.T).reshape(FLAT)
    def bf16r_host(v):
        u = lax.bitcast_convert_type(v, jnp.int32)
        u = u + 0x7FFF + jnp.bitwise_and(jnp.right_shift(u, 16), 1)
        u = jnp.bitwise_and(u, jnp.int32(-65536))
        return lax.bitcast_convert_type(u, f32)

    wfr = bf16r_host(Wf)
    wsr = bf16r_host(Ws)
    wfb = jnp.broadcast_to(wfr[:, :, None], (C, ZD, 16)).reshape(-1)
    wsb = jnp.broadcast_to(wsr[:, :, None], (C, ZD, 16)).reshape(-1)
    bfb = jnp.broadcast_to(bf.reshape(C, 1), (C, 16)).reshape(-1)
    bsb = jnp.broadcast_to(bs.reshape(C, 1), (C, 16)).reshape(-1)
    wl2 = jnp.zeros((C, NPAD), f32).at[:, :N].set(Wl[0].reshape(N, C).T).reshape(FLAT)
    blb = jnp.zeros((16,), f32).at[0].set(bl[0])
    par = jnp.concatenate([xt, wfb, wsb, bfb, bsb, wl2, blb])

    srcp = jnp.zeros((EPAD,), jnp.int32).at[:E].set(edge_index[0]).reshape(NSUB, 1, EPS)
    dstp = jnp.full((EPAD,), N, jnp.int32).at[:E].set(edge_index[1]).reshape(NSUB, 1, EPS)
    attr_t = jnp.zeros((D, EPAD), f32).at[:, :E].set(edge_attr.T)
    attr_i = lax.bitcast_convert_type(attr_t, jnp.int32)
    attr_b = attr_i.reshape(D, NSUB, EPS).transpose(1, 0, 2)
    edg = jnp.concatenate([srcp, dstp, attr_b], axis=1).reshape(-1)

    out16 = _run(par, edg)
    return out16[15:16]
